# baseline (device time: 329266 ns/iter reference)
import jax
import jax.numpy as jnp
from jax import lax
from jax.experimental import pallas as pl
from jax.experimental.pallas import tpu as pltpu

P = 32
M = 1536
N = 1536
CR = M // P


def kernel(A, B):
    def body(a_ref, b_ref, out_ref, rs_buf,
             rs_send_sems, rs_recv_sems, ag_send_sems, ag_recv_sems):
        my = lax.axis_index("i")
        left = lax.rem(my + P - 1, P)
        right = lax.rem(my + 1, P)

        barrier = pltpu.get_barrier_semaphore()
        for nbr in (left, right):
            pl.semaphore_signal(
                barrier, inc=1,
                device_id=(nbr,), device_id_type=pl.DeviceIdType.MESH,
            )
        pl.semaphore_wait(barrier, 2)

        out_ref[...] = jnp.dot(
            a_ref[...].astype(jnp.bfloat16),
            b_ref[...].astype(jnp.bfloat16),
            preferred_element_type=jnp.float32,
        )

        for s in range(P - 1):
            c_send = lax.rem(my - s + P, P)
            c_recv = lax.rem(my - s - 1 + P, P)
            rdma = pltpu.make_async_remote_copy(
                src_ref=out_ref.at[pl.ds(c_send * CR, CR)],
                dst_ref=rs_buf.at[s],
                send_sem=rs_send_sems.at[s],
                recv_sem=rs_recv_sems.at[s],
                device_id=(right,),
                device_id_type=pl.DeviceIdType.MESH,
            )
            rdma.start()
            rdma.wait()
            idx = pl.ds(c_recv * CR, CR)
            out_ref[idx] = out_ref[idx] + rs_buf[s]

        for t in range(P - 1):
            c = lax.rem(my + 1 - t + P, P)
            rdma = pltpu.make_async_remote_copy(
                src_ref=out_ref.at[pl.ds(c * CR, CR)],
                dst_ref=out_ref.at[pl.ds(c * CR, CR)],
                send_sem=ag_send_sems.at[t],
                recv_sem=ag_recv_sems.at[t],
                device_id=(right,),
                device_id_type=pl.DeviceIdType.MESH,
            )
            rdma.start()
            rdma.wait()

        out_ref[...] = jnp.maximum(out_ref[...], 0.0)

    return pl.pallas_call(
        body,
        out_shape=jax.ShapeDtypeStruct((M, N), jnp.float32),
        in_specs=[
            pl.BlockSpec(memory_space=pltpu.VMEM),
            pl.BlockSpec(memory_space=pltpu.VMEM),
        ],
        out_specs=pl.BlockSpec(memory_space=pltpu.VMEM),
        scratch_shapes=[
            pltpu.VMEM((P - 1, CR, N), jnp.float32),
            pltpu.SemaphoreType.DMA((P - 1,)),
            pltpu.SemaphoreType.DMA((P - 1,)),
            pltpu.SemaphoreType.DMA((P - 1,)),
            pltpu.SemaphoreType.DMA((P - 1,)),
        ],
        compiler_params=pltpu.CompilerParams(collective_id=0),
    )(A, B)


# device time: 155164 ns/iter; 2.1221x vs baseline; 2.1221x over previous
import jax
import jax.numpy as jnp
from jax import lax
from jax.experimental import pallas as pl
from jax.experimental.pallas import tpu as pltpu

P = 32
R = 8
C = 4
M = 1536
N = 1536
JR = M // R
SR = JR // C

_MESH = pl.DeviceIdType.MESH


def kernel(A, B):
    def body(a_ref, b_ref, out_ref,
             p1_land, p1_stage, p2a_land, p2a_stage,
             p2b_land, p2b_stage, p3_land, p3_stage,
             p1_ssem, p1_rsem, p2a_ssem, p2a_rsem,
             p2b_ssem, p2b_rsem, p3_ssem, p3_rsem):
        my = lax.axis_index("i")
        r = my // R
        q = my % R
        row_right = r * R + lax.rem(q + 1, R)
        row_left = r * R + lax.rem(q + R - 1, R)
        col_next = lax.rem(r + 1, C) * R + q
        col_prev = lax.rem(r + C - 1, C) * R + q

        barrier = pltpu.get_barrier_semaphore()
        for nbr in (row_left, row_right, col_prev, col_next):
            pl.semaphore_signal(
                barrier, inc=1, device_id=(nbr,), device_id_type=_MESH,
            )
        pl.semaphore_wait(barrier, 4)

        out_ref[...] = jnp.dot(
            a_ref[...].astype(jnp.bfloat16),
            b_ref[...].astype(jnp.bfloat16),
            preferred_element_type=jnp.float32,
        )

        for s in range(R - 1):
            j_send = lax.rem(q - s + R, R)
            j_recv = lax.rem(q - s - 1 + R, R)
            p1_stage[...] = out_ref[pl.ds(j_send * JR, JR)].astype(jnp.bfloat16)
            rdma = pltpu.make_async_remote_copy(
                src_ref=p1_stage,
                dst_ref=p1_land.at[s],
                send_sem=p1_ssem.at[s],
                recv_sem=p1_rsem.at[s],
                device_id=(row_right,),
                device_id_type=_MESH,
            )
            rdma.start()
            rdma.wait()
            idx = pl.ds(j_recv * JR, JR)
            out_ref[idx] = out_ref[idx] + p1_land[s].astype(jnp.float32)

        j_own = lax.rem(q + 1, R)
        base = j_own * JR

        for s in range(C - 1):
            c_send = lax.rem(r - s + C, C)
            c_recv = lax.rem(r - s - 1 + C, C)
            p2a_stage[...] = out_ref[
                pl.ds(base + c_send * SR, SR)].astype(jnp.bfloat16)
            rdma = pltpu.make_async_remote_copy(
                src_ref=p2a_stage,
                dst_ref=p2a_land.at[s],
                send_sem=p2a_ssem.at[s],
                recv_sem=p2a_rsem.at[s],
                device_id=(col_next,),
                device_id_type=_MESH,
            )
            rdma.start()
            rdma.wait()
            idx = pl.ds(base + c_recv * SR, SR)
            out_ref[idx] = out_ref[idx] + p2a_land[s].astype(jnp.float32)

        c_own = lax.rem(r + 1, C)

        p2b_stage[...] = out_ref[pl.ds(base + c_own * SR, SR)].astype(jnp.bfloat16)
        for t in range(C - 1):
            c_recv = lax.rem(r - t + C, C)
            rdma = pltpu.make_async_remote_copy(
                src_ref=p2b_stage if t == 0 else p2b_land.at[t - 1],
                dst_ref=p2b_land.at[t],
                send_sem=p2b_ssem.at[t],
                recv_sem=p2b_rsem.at[t],
                device_id=(col_next,),
                device_id_type=_MESH,
            )
            rdma.start()
            rdma.wait()
            out_ref[pl.ds(base + c_recv * SR, SR)] = (
                p2b_land[t].astype(jnp.float32))

        p3_stage[...] = out_ref[pl.ds(base, JR)].astype(jnp.bfloat16)
        for t in range(R - 1):
            j_recv = lax.rem(q - t + R, R)
            rdma = pltpu.make_async_remote_copy(
                src_ref=p3_stage if t == 0 else p3_land.at[t - 1],
                dst_ref=p3_land.at[t],
                send_sem=p3_ssem.at[t],
                recv_sem=p3_rsem.at[t],
                device_id=(row_right,),
                device_id_type=_MESH,
            )
            rdma.start()
            rdma.wait()
            out_ref[pl.ds(j_recv * JR, JR)] = p3_land[t].astype(jnp.float32)

        out_ref[...] = jnp.maximum(out_ref[...], 0.0)

    return pl.pallas_call(
        body,
        out_shape=jax.ShapeDtypeStruct((M, N), jnp.float32),
        in_specs=[
            pl.BlockSpec(memory_space=pltpu.VMEM),
            pl.BlockSpec(memory_space=pltpu.VMEM),
        ],
        out_specs=pl.BlockSpec(memory_space=pltpu.VMEM),
        scratch_shapes=[
            pltpu.VMEM((R - 1, JR, N), jnp.bfloat16),
            pltpu.VMEM((JR, N), jnp.bfloat16),
            pltpu.VMEM((C - 1, SR, N), jnp.bfloat16),
            pltpu.VMEM((SR, N), jnp.bfloat16),
            pltpu.VMEM((C - 1, SR, N), jnp.bfloat16),
            pltpu.VMEM((SR, N), jnp.bfloat16),
            pltpu.VMEM((R - 1, JR, N), jnp.bfloat16),
            pltpu.VMEM((JR, N), jnp.bfloat16),
            pltpu.SemaphoreType.DMA((R - 1,)),
            pltpu.SemaphoreType.DMA((R - 1,)),
            pltpu.SemaphoreType.DMA((C - 1,)),
            pltpu.SemaphoreType.DMA((C - 1,)),
            pltpu.SemaphoreType.DMA((C - 1,)),
            pltpu.SemaphoreType.DMA((C - 1,)),
            pltpu.SemaphoreType.DMA((R - 1,)),
            pltpu.SemaphoreType.DMA((R - 1,)),
        ],
        compiler_params=pltpu.CompilerParams(collective_id=0),
    )(A, B)


# device time: 147367 ns/iter; 2.2343x vs baseline; 1.0529x over previous
import jax
import jax.numpy as jnp
from jax import lax
from jax.experimental import pallas as pl
from jax.experimental.pallas import tpu as pltpu

P = 32
R = 8
C = 4
M = 1536
N = 1536
NH = N // 2
JR = M // R
SR = JR // C

_MESH = pl.DeviceIdType.MESH


def kernel(A, B):
    def body(a_ref, b_ref, out_ref,
             p1p_land, p1p_stage, p1m_land, p1m_stage,
             p2ap_land, p2ap_stage, p2am_land, p2am_stage,
             p2bp_land, p2bp_stage, p2bm_land, p2bm_stage,
             p3p_land, p3p_stage, p3m_land, p3m_stage,
             p1p_ss, p1p_rs, p1m_ss, p1m_rs,
             p2ap_ss, p2ap_rs, p2am_ss, p2am_rs,
             p2bp_ss, p2bp_rs, p2bm_ss, p2bm_rs,
             p3p_ss, p3p_rs, p3m_ss, p3m_rs):
        my = lax.axis_index("i")
        r = my // R
        q = my % R
        row_right = r * R + lax.rem(q + 1, R)
        row_left = r * R + lax.rem(q + R - 1, R)
        col_next = lax.rem(r + 1, C) * R + q
        col_prev = lax.rem(r + C - 1, C) * R + q

        lo = slice(0, NH)
        hi = slice(NH, N)

        barrier = pltpu.get_barrier_semaphore()
        for nbr in (row_left, row_right, col_prev, col_next):
            pl.semaphore_signal(
                barrier, inc=1, device_id=(nbr,), device_id_type=_MESH,
            )
        pl.semaphore_wait(barrier, 4)

        out_ref[...] = jnp.dot(
            a_ref[...].astype(jnp.bfloat16),
            b_ref[...].astype(jnp.bfloat16),
            preferred_element_type=jnp.float32,
        )

        def copy(src, dst, ssem, rsem, dev):
            return pltpu.make_async_remote_copy(
                src_ref=src, dst_ref=dst, send_sem=ssem, recv_sem=rsem,
                device_id=(dev,), device_id_type=_MESH,
            )

        for s in range(R - 1):
            jp_send = lax.rem(q - s + R, R)
            jp_recv = lax.rem(q - s - 1 + R, R)
            jm_send = lax.rem(q + s, R)
            jm_recv = lax.rem(q + s + 1, R)
            p1p_stage[...] = out_ref[pl.ds(jp_send * JR, JR), lo].astype(jnp.bfloat16)
            p1m_stage[...] = out_ref[pl.ds(jm_send * JR, JR), hi].astype(jnp.bfloat16)
            rp = copy(p1p_stage, p1p_land.at[s], p1p_ss.at[s], p1p_rs.at[s], row_right)
            rm = copy(p1m_stage, p1m_land.at[s], p1m_ss.at[s], p1m_rs.at[s], row_left)
            rp.start()
            rm.start()
            rp.wait()
            rm.wait()
            ip = pl.ds(jp_recv * JR, JR)
            im = pl.ds(jm_recv * JR, JR)
            out_ref[ip, lo] = out_ref[ip, lo] + p1p_land[s].astype(jnp.float32)
            out_ref[im, hi] = out_ref[im, hi] + p1m_land[s].astype(jnp.float32)

        jp_own = lax.rem(q + 1, R)
        jm_own = lax.rem(q + R - 1, R)
        base_p = jp_own * JR
        base_m = jm_own * JR

        for s in range(C - 1):
            cp_send = lax.rem(r - s + C, C)
            cp_recv = lax.rem(r - s - 1 + C, C)
            cm_send = lax.rem(r + s, C)
            cm_recv = lax.rem(r + s + 1, C)
            p2ap_stage[...] = out_ref[
                pl.ds(base_p + cp_send * SR, SR), lo].astype(jnp.bfloat16)
            p2am_stage[...] = out_ref[
                pl.ds(base_m + cm_send * SR, SR), hi].astype(jnp.bfloat16)
            rp = copy(p2ap_stage, p2ap_land.at[s], p2ap_ss.at[s], p2ap_rs.at[s], col_next)
            rm = copy(p2am_stage, p2am_land.at[s], p2am_ss.at[s], p2am_rs.at[s], col_prev)
            rp.start()
            rm.start()
            rp.wait()
            rm.wait()
            ip = pl.ds(base_p + cp_recv * SR, SR)
            im = pl.ds(base_m + cm_recv * SR, SR)
            out_ref[ip, lo] = out_ref[ip, lo] + p2ap_land[s].astype(jnp.float32)
            out_ref[im, hi] = out_ref[im, hi] + p2am_land[s].astype(jnp.float32)

        cp_own = lax.rem(r + 1, C)
        cm_own = lax.rem(r + C - 1, C)

        p2bp_stage[...] = out_ref[
            pl.ds(base_p + cp_own * SR, SR), lo].astype(jnp.bfloat16)
        p2bm_stage[...] = out_ref[
            pl.ds(base_m + cm_own * SR, SR), hi].astype(jnp.bfloat16)
        for t in range(C - 1):
            cp_recv = lax.rem(r - t + C, C)
            cm_recv = lax.rem(r + t, C)
            rp = copy(p2bp_stage if t == 0 else p2bp_land.at[t - 1],
                      p2bp_land.at[t], p2bp_ss.at[t], p2bp_rs.at[t], col_next)
            rm = copy(p2bm_stage if t == 0 else p2bm_land.at[t - 1],
                      p2bm_land.at[t], p2bm_ss.at[t], p2bm_rs.at[t], col_prev)
            rp.start()
            rm.start()
            rp.wait()
            rm.wait()
            out_ref[pl.ds(base_p + cp_recv * SR, SR), lo] = (
                p2bp_land[t].astype(jnp.float32))
            out_ref[pl.ds(base_m + cm_recv * SR, SR), hi] = (
                p2bm_land[t].astype(jnp.float32))

        p3p_stage[...] = out_ref[pl.ds(base_p, JR), lo].astype(jnp.bfloat16)
        p3m_stage[...] = out_ref[pl.ds(base_m, JR), hi].astype(jnp.bfloat16)
        for t in range(R - 1):
            jp_recv = lax.rem(q - t + R, R)
            jm_recv = lax.rem(q + t, R)
            rp = copy(p3p_stage if t == 0 else p3p_land.at[t - 1],
                      p3p_land.at[t], p3p_ss.at[t], p3p_rs.at[t], row_right)
            rm = copy(p3m_stage if t == 0 else p3m_land.at[t - 1],
                      p3m_land.at[t], p3m_ss.at[t], p3m_rs.at[t], row_left)
            rp.start()
            rm.start()
            rp.wait()
            rm.wait()
            out_ref[pl.ds(jp_recv * JR, JR), lo] = p3p_land[t].astype(jnp.float32)
            out_ref[pl.ds(jm_recv * JR, JR), hi] = p3m_land[t].astype(jnp.float32)

        out_ref[...] = jnp.maximum(out_ref[...], 0.0)

    return pl.pallas_call(
        body,
        out_shape=jax.ShapeDtypeStruct((M, N), jnp.float32),
        in_specs=[
            pl.BlockSpec(memory_space=pltpu.VMEM),
            pl.BlockSpec(memory_space=pltpu.VMEM),
        ],
        out_specs=pl.BlockSpec(memory_space=pltpu.VMEM),
        scratch_shapes=[
            pltpu.VMEM((R - 1, JR, NH), jnp.bfloat16),
            pltpu.VMEM((JR, NH), jnp.bfloat16),
            pltpu.VMEM((R - 1, JR, NH), jnp.bfloat16),
            pltpu.VMEM((JR, NH), jnp.bfloat16),
            pltpu.VMEM((C - 1, SR, NH), jnp.bfloat16),
            pltpu.VMEM((SR, NH), jnp.bfloat16),
            pltpu.VMEM((C - 1, SR, NH), jnp.bfloat16),
            pltpu.VMEM((SR, NH), jnp.bfloat16),
            pltpu.VMEM((C - 1, SR, NH), jnp.bfloat16),
            pltpu.VMEM((SR, NH), jnp.bfloat16),
            pltpu.VMEM((C - 1, SR, NH), jnp.bfloat16),
            pltpu.VMEM((SR, NH), jnp.bfloat16),
            pltpu.VMEM((R - 1, JR, NH), jnp.bfloat16),
            pltpu.VMEM((JR, NH), jnp.bfloat16),
            pltpu.VMEM((R - 1, JR, NH), jnp.bfloat16),
            pltpu.VMEM((JR, NH), jnp.bfloat16),
            pltpu.SemaphoreType.DMA((R - 1,)),
            pltpu.SemaphoreType.DMA((R - 1,)),
            pltpu.SemaphoreType.DMA((R - 1,)),
            pltpu.SemaphoreType.DMA((R - 1,)),
            pltpu.SemaphoreType.DMA((C - 1,)),
            pltpu.SemaphoreType.DMA((C - 1,)),
            pltpu.SemaphoreType.DMA((C - 1,)),
            pltpu.SemaphoreType.DMA((C - 1,)),
            pltpu.SemaphoreType.DMA((C - 1,)),
            pltpu.SemaphoreType.DMA((C - 1,)),
            pltpu.SemaphoreType.DMA((C - 1,)),
            pltpu.SemaphoreType.DMA((C - 1,)),
            pltpu.SemaphoreType.DMA((R - 1,)),
            pltpu.SemaphoreType.DMA((R - 1,)),
            pltpu.SemaphoreType.DMA((R - 1,)),
            pltpu.SemaphoreType.DMA((R - 1,)),
        ],
        compiler_params=pltpu.CompilerParams(collective_id=0),
    )(A, B)


# device time: 113696 ns/iter; 2.8960x vs baseline; 1.2961x over previous
import jax
import jax.numpy as jnp
from jax import lax
from jax.experimental import pallas as pl
from jax.experimental.pallas import tpu as pltpu

P = 32
R = 8
C = 4
M = 1536
N = 1536
NH = N // 2
JR = M // R
SR = JR // C

_MESH = pl.DeviceIdType.MESH
_BF16 = jnp.bfloat16
_F32 = jnp.float32


def _q_of(v):
    y = jnp.where(v == 0, 0, jnp.where(v <= 4, v - 1, 8 - v))
    x = jnp.where((v >= 1) & (v <= 4), 1, 0)
    return 2 * y + lax.rem(x + y, 2)


def _k_of(q):
    y = q // 2
    x = lax.rem(q + y, 2)
    return jnp.where(x == 1, y + 1, jnp.where(y == 0, 0, 8 - y))


def kernel(A, B):
    def body(a_ref, b_ref, out_ref,
             p1p_land, p1p_stage, p1m_land, p1m_stage,
             p2ap_land, p2ap_stage, p2am_land, p2am_stage,
             p2bp_land, p2bp_stage, p2bm_land, p2bm_stage,
             p3p_land, p3p_stage, p3m_land, p3m_stage,
             p1p_ss, p1p_rs, p1m_ss, p1m_rs,
             p2ap_ss, p2ap_rs, p2am_ss, p2am_rs,
             p2bp_ss, p2bp_rs, p2bm_ss, p2bm_rs,
             p3p_ss, p3p_rs, p3m_ss, p3m_rs):
        my = lax.axis_index("i")
        r = my // R
        q = my % R
        k = _k_of(q)
        row_right = r * R + _q_of(lax.rem(k + 1, R))
        row_left = r * R + _q_of(lax.rem(k + R - 1, R))
        col_next = lax.rem(r + 1, C) * R + q
        col_prev = lax.rem(r + C - 1, C) * R + q

        lo = slice(0, NH)
        hi = slice(NH, N)

        barrier = pltpu.get_barrier_semaphore()
        for nbr in (row_left, row_right, col_prev, col_next):
            pl.semaphore_signal(
                barrier, inc=1, device_id=(nbr,), device_id_type=_MESH,
            )
        pl.semaphore_wait(barrier, 4)

        out_ref[...] = jnp.dot(
            a_ref[...].astype(_BF16),
            b_ref[...].astype(_BF16),
            preferred_element_type=_F32,
        )

        def copy(src, dst, ssem, rsem, dev):
            return pltpu.make_async_remote_copy(
                src_ref=src, dst_ref=dst, send_sem=ssem, recv_sem=rsem,
                device_id=(dev,), device_id_type=_MESH,
            )

        p1p_stage[...] = out_ref[pl.ds(q * JR, JR), lo].astype(_BF16)
        p1m_stage[...] = out_ref[pl.ds(q * JR, JR), hi].astype(_BF16)
        for s in range(R - 1):
            rp = copy(p1p_stage, p1p_land.at[s], p1p_ss.at[s], p1p_rs.at[s],
                      row_right)
            rm = copy(p1m_stage, p1m_land.at[s], p1m_ss.at[s], p1m_rs.at[s],
                      row_left)
            rp.start()
            rm.start()
            rp.wait()
            rm.wait()
            jp_recv = _q_of(lax.rem(k - s - 1 + R, R))
            jm_recv = _q_of(lax.rem(k + s + 1, R))
            ip = pl.ds(jp_recv * JR, JR)
            im = pl.ds(jm_recv * JR, JR)
            new_p = out_ref[ip, lo] + p1p_land[s].astype(_F32)
            new_m = out_ref[im, hi] + p1m_land[s].astype(_F32)
            out_ref[ip, lo] = new_p
            out_ref[im, hi] = new_m
            if s < R - 2:
                p1p_stage[...] = new_p.astype(_BF16)
                p1m_stage[...] = new_m.astype(_BF16)

        jp_own = _q_of(lax.rem(k + 1, R))
        jm_own = _q_of(lax.rem(k + R - 1, R))
        base_p = jp_own * JR
        base_m = jm_own * JR

        p2ap_stage[...] = out_ref[pl.ds(base_p + r * SR, SR), lo].astype(_BF16)
        p2am_stage[...] = out_ref[pl.ds(base_m + r * SR, SR), hi].astype(_BF16)
        for s in range(C - 1):
            rp = copy(p2ap_stage, p2ap_land.at[s], p2ap_ss.at[s],
                      p2ap_rs.at[s], col_next)
            rm = copy(p2am_stage, p2am_land.at[s], p2am_ss.at[s],
                      p2am_rs.at[s], col_prev)
            rp.start()
            rm.start()
            rp.wait()
            rm.wait()
            cp_recv = lax.rem(r - s - 1 + C, C)
            cm_recv = lax.rem(r + s + 1, C)
            ip = pl.ds(base_p + cp_recv * SR, SR)
            im = pl.ds(base_m + cm_recv * SR, SR)
            new_p = out_ref[ip, lo] + p2ap_land[s].astype(_F32)
            new_m = out_ref[im, hi] + p2am_land[s].astype(_F32)
            out_ref[ip, lo] = new_p
            out_ref[im, hi] = new_m
            if s < C - 2:
                p2ap_stage[...] = new_p.astype(_BF16)
                p2am_stage[...] = new_m.astype(_BF16)

        cp_own = lax.rem(r + 1, C)
        cm_own = lax.rem(r + C - 1, C)

        p2bp_stage[...] = out_ref[
            pl.ds(base_p + cp_own * SR, SR), lo].astype(_BF16)
        p2bm_stage[...] = out_ref[
            pl.ds(base_m + cm_own * SR, SR), hi].astype(_BF16)
        for t in range(C - 1):
            rp = copy(p2bp_stage if t == 0 else p2bp_land.at[t - 1],
                      p2bp_land.at[t], p2bp_ss.at[t], p2bp_rs.at[t], col_next)
            rm = copy(p2bm_stage if t == 0 else p2bm_land.at[t - 1],
                      p2bm_land.at[t], p2bm_ss.at[t], p2bm_rs.at[t], col_prev)
            rp.start()
            rm.start()
            rp.wait()
            rm.wait()
            cp_recv = lax.rem(r - t + C, C)
            cm_recv = lax.rem(r + t, C)
            out_ref[pl.ds(base_p + cp_recv * SR, SR), lo] = (
                p2bp_land[t].astype(_F32))
            out_ref[pl.ds(base_m + cm_recv * SR, SR), hi] = (
                p2bm_land[t].astype(_F32))

        p3p_stage[...] = out_ref[pl.ds(base_p, JR), lo].astype(_BF16)
        p3m_stage[...] = out_ref[pl.ds(base_m, JR), hi].astype(_BF16)
        for t in range(R - 1):
            rp = copy(p3p_stage if t == 0 else p3p_land.at[t - 1],
                      p3p_land.at[t], p3p_ss.at[t], p3p_rs.at[t], row_right)
            rm = copy(p3m_stage if t == 0 else p3m_land.at[t - 1],
                      p3m_land.at[t], p3m_ss.at[t], p3m_rs.at[t], row_left)
            rp.start()
            rm.start()
            rp.wait()
            rm.wait()
            jp_recv = _q_of(lax.rem(k - t + R, R))
            jm_recv = _q_of(lax.rem(k + t, R))
            out_ref[pl.ds(jp_recv * JR, JR), lo] = p3p_land[t].astype(_F32)
            out_ref[pl.ds(jm_recv * JR, JR), hi] = p3m_land[t].astype(_F32)

        out_ref[...] = jnp.maximum(out_ref[...], 0.0)

    return pl.pallas_call(
        body,
        out_shape=jax.ShapeDtypeStruct((M, N), jnp.float32),
        in_specs=[
            pl.BlockSpec(memory_space=pltpu.VMEM),
            pl.BlockSpec(memory_space=pltpu.VMEM),
        ],
        out_specs=pl.BlockSpec(memory_space=pltpu.VMEM),
        scratch_shapes=[
            pltpu.VMEM((R - 1, JR, NH), _BF16),
            pltpu.VMEM((JR, NH), _BF16),
            pltpu.VMEM((R - 1, JR, NH), _BF16),
            pltpu.VMEM((JR, NH), _BF16),
            pltpu.VMEM((C - 1, SR, NH), _BF16),
            pltpu.VMEM((SR, NH), _BF16),
            pltpu.VMEM((C - 1, SR, NH), _BF16),
            pltpu.VMEM((SR, NH), _BF16),
            pltpu.VMEM((C - 1, SR, NH), _BF16),
            pltpu.VMEM((SR, NH), _BF16),
            pltpu.VMEM((C - 1, SR, NH), _BF16),
            pltpu.VMEM((SR, NH), _BF16),
            pltpu.VMEM((R - 1, JR, NH), _BF16),
            pltpu.VMEM((JR, NH), _BF16),
            pltpu.VMEM((R - 1, JR, NH), _BF16),
            pltpu.VMEM((JR, NH), _BF16),
            pltpu.SemaphoreType.DMA((R - 1,)),
            pltpu.SemaphoreType.DMA((R - 1,)),
            pltpu.SemaphoreType.DMA((R - 1,)),
            pltpu.SemaphoreType.DMA((R - 1,)),
            pltpu.SemaphoreType.DMA((C - 1,)),
            pltpu.SemaphoreType.DMA((C - 1,)),
            pltpu.SemaphoreType.DMA((C - 1,)),
            pltpu.SemaphoreType.DMA((C - 1,)),
            pltpu.SemaphoreType.DMA((C - 1,)),
            pltpu.SemaphoreType.DMA((C - 1,)),
            pltpu.SemaphoreType.DMA((C - 1,)),
            pltpu.SemaphoreType.DMA((C - 1,)),
            pltpu.SemaphoreType.DMA((R - 1,)),
            pltpu.SemaphoreType.DMA((R - 1,)),
            pltpu.SemaphoreType.DMA((R - 1,)),
            pltpu.SemaphoreType.DMA((R - 1,)),
        ],
        compiler_params=pltpu.CompilerParams(collective_id=0),
    )(A, B)


# device time: 103554 ns/iter; 3.1797x vs baseline; 1.0979x over previous
import jax
import jax.numpy as jnp
from jax import lax
from jax.experimental import pallas as pl
from jax.experimental.pallas import tpu as pltpu

P = 32
R = 8
C = 4
M = 1536
N = 1536
NH = N // 2
JR = M // R
SR = JR // C

_MESH = pl.DeviceIdType.MESH
_BF16 = jnp.bfloat16
_F32 = jnp.float32


def _q_of(v):
    y = jnp.where(v == 0, 0, jnp.where(v <= 4, v - 1, 8 - v))
    x = jnp.where((v >= 1) & (v <= 4), 1, 0)
    return 2 * y + lax.rem(x + y, 2)


def _k_of(q):
    y = q // 2
    x = lax.rem(q + y, 2)
    return jnp.where(x == 1, y + 1, jnp.where(y == 0, 0, 8 - y))


def kernel(A, B):
    def body(a_ref, b_ref, out_ref, b_bf,
             p1p_land, p1p_stage, p1m_land, p1m_stage,
             p2ap_land, p2ap_stage, p2am_land, p2am_stage,
             p2bp_land, p2bp_stage, p2bm_land, p2bm_stage,
             p3p_land, p3p_stage, p3m_land, p3m_stage,
             p1p_ss, p1p_rs, p1m_ss, p1m_rs,
             p2ap_ss, p2ap_rs, p2am_ss, p2am_rs,
             p2bp_ss, p2bp_rs, p2bm_ss, p2bm_rs,
             p3p_ss, p3p_rs, p3m_ss, p3m_rs):
        my = lax.axis_index("i")
        r = my // R
        q = my % R
        k = _k_of(q)
        row_right = r * R + _q_of(lax.rem(k + 1, R))
        row_left = r * R + _q_of(lax.rem(k + R - 1, R))
        col_next = lax.rem(r + 1, C) * R + q
        col_prev = lax.rem(r + C - 1, C) * R + q

        lo = slice(0, NH)
        hi = slice(NH, N)

        barrier = pltpu.get_barrier_semaphore()
        for nbr in (row_left, row_right, col_prev, col_next):
            pl.semaphore_signal(
                barrier, inc=1, device_id=(nbr,), device_id_type=_MESH,
            )
        pl.semaphore_wait(barrier, 4)

        def copy(src, dst, ssem, rsem, dev):
            return pltpu.make_async_remote_copy(
                src_ref=src, dst_ref=dst, send_sem=ssem, recv_sem=rsem,
                device_id=(dev,), device_id_type=_MESH,
            )

        def p1_desc(s, plus):
            if plus:
                return copy(p1p_stage, p1p_land.at[s], p1p_ss.at[s],
                            p1p_rs.at[s], row_right)
            return copy(p1m_stage, p1m_land.at[s], p1m_ss.at[s],
                        p1m_rs.at[s], row_left)

        def p2a_desc(s, plus):
            if plus:
                return copy(p2ap_stage, p2ap_land.at[s], p2ap_ss.at[s],
                            p2ap_rs.at[s], col_next)
            return copy(p2am_stage, p2am_land.at[s], p2am_ss.at[s],
                        p2am_rs.at[s], col_prev)

        def p2b_desc(t, plus):
            if plus:
                return copy(p2bp_stage if t == 0 else p2bp_land.at[t - 1],
                            p2bp_land.at[t], p2bp_ss.at[t], p2bp_rs.at[t],
                            col_next)
            return copy(p2bm_stage if t == 0 else p2bm_land.at[t - 1],
                        p2bm_land.at[t], p2bm_ss.at[t], p2bm_rs.at[t],
                        col_prev)

        def p3_desc(t, plus):
            if plus:
                return copy(p3p_stage if t == 0 else p3p_land.at[t - 1],
                            p3p_land.at[t], p3p_ss.at[t], p3p_rs.at[t],
                            row_right)
            return copy(p3m_stage if t == 0 else p3m_land.at[t - 1],
                        p3m_land.at[t], p3m_ss.at[t], p3m_rs.at[t],
                        row_left)

        b_bf[...] = b_ref[...].astype(_BF16)
        ds_q = pl.ds(q * JR, JR)
        out_ref[ds_q, :] = jnp.dot(
            a_ref[ds_q, :].astype(_BF16), b_bf[...],
            preferred_element_type=_F32,
        )
        p1p_stage[...] = out_ref[ds_q, lo].astype(_BF16)
        p1m_stage[...] = out_ref[ds_q, hi].astype(_BF16)
        p1_desc(0, True).start()
        p1_desc(0, False).start()
        for d in range(1, R):
            j = lax.rem(q + d, R)
            ds_j = pl.ds(j * JR, JR)
            out_ref[ds_j, :] = jnp.dot(
                a_ref[ds_j, :].astype(_BF16), b_bf[...],
                preferred_element_type=_F32,
            )

        for s in range(R - 1):
            for plus in (True, False):
                d = p1_desc(s, plus)
                d.wait_recv()
                if plus:
                    jj = _q_of(lax.rem(k - s - 1 + R, R))
                    idx, half, land = pl.ds(jj * JR, JR), lo, p1p_land
                    stage = p1p_stage
                else:
                    jj = _q_of(lax.rem(k + s + 1, R))
                    idx, half, land = pl.ds(jj * JR, JR), hi, p1m_land
                    stage = p1m_stage
                new = out_ref[idx, half] + land[s].astype(_F32)
                out_ref[idx, half] = new
                d.wait_send()
                if s < R - 2:
                    stage[...] = new.astype(_BF16)
                    p1_desc(s + 1, plus).start()

        jp_own = _q_of(lax.rem(k + 1, R))
        jm_own = _q_of(lax.rem(k + R - 1, R))
        base_p = jp_own * JR
        base_m = jm_own * JR

        p2ap_stage[...] = out_ref[pl.ds(base_p + r * SR, SR), lo].astype(_BF16)
        p2am_stage[...] = out_ref[pl.ds(base_m + r * SR, SR), hi].astype(_BF16)
        p2a_desc(0, True).start()
        p2a_desc(0, False).start()
        for s in range(C - 1):
            for plus in (True, False):
                d = p2a_desc(s, plus)
                d.wait_recv()
                if plus:
                    cc = lax.rem(r - s - 1 + C, C)
                    idx = pl.ds(base_p + cc * SR, SR)
                    half, land, stage = lo, p2ap_land, p2ap_stage
                else:
                    cc = lax.rem(r + s + 1, C)
                    idx = pl.ds(base_m + cc * SR, SR)
                    half, land, stage = hi, p2am_land, p2am_stage
                new = out_ref[idx, half] + land[s].astype(_F32)
                out_ref[idx, half] = new
                d.wait_send()
                if s < C - 2:
                    stage[...] = new.astype(_BF16)
                    p2a_desc(s + 1, plus).start()

        cp_own = lax.rem(r + 1, C)
        cm_own = lax.rem(r + C - 1, C)

        p2bp_stage[...] = out_ref[
            pl.ds(base_p + cp_own * SR, SR), lo].astype(_BF16)
        p2bm_stage[...] = out_ref[
            pl.ds(base_m + cm_own * SR, SR), hi].astype(_BF16)
        p2b_desc(0, True).start()
        p2b_desc(0, False).start()
        for t in range(C - 1):
            for plus in (True, False):
                d = p2b_desc(t, plus)
                d.wait_recv()
                if t < C - 2:
                    p2b_desc(t + 1, plus).start()
                if plus:
                    cc = lax.rem(r - t + C, C)
                    out_ref[pl.ds(base_p + cc * SR, SR), lo] = (
                        p2bp_land[t].astype(_F32))
                else:
                    cc = lax.rem(r + t, C)
                    out_ref[pl.ds(base_m + cc * SR, SR), hi] = (
                        p2bm_land[t].astype(_F32))

        p3p_stage[...] = out_ref[pl.ds(base_p, JR), lo].astype(_BF16)
        p3m_stage[...] = out_ref[pl.ds(base_m, JR), hi].astype(_BF16)
        p3_desc(0, True).start()
        p3_desc(0, False).start()
        for t in range(R - 1):
            for plus in (True, False):
                d = p3_desc(t, plus)
                d.wait_recv()
                if t < R - 2:
                    p3_desc(t + 1, plus).start()
                if plus:
                    jj = _q_of(lax.rem(k - t + R, R))
                    out_ref[pl.ds(jj * JR, JR), lo] = jnp.maximum(
                        p3p_land[t].astype(_F32), 0.0)
                else:
                    jj = _q_of(lax.rem(k + t, R))
                    out_ref[pl.ds(jj * JR, JR), hi] = jnp.maximum(
                        p3m_land[t].astype(_F32), 0.0)

        out_ref[pl.ds(base_p, JR), lo] = jnp.maximum(
            out_ref[pl.ds(base_p, JR), lo], 0.0)
        out_ref[pl.ds(base_m, JR), hi] = jnp.maximum(
            out_ref[pl.ds(base_m, JR), hi], 0.0)

        for t in range(C - 1):
            p2b_desc(t, True).wait_send()
            p2b_desc(t, False).wait_send()
        for t in range(R - 1):
            p3_desc(t, True).wait_send()
            p3_desc(t, False).wait_send()

    return pl.pallas_call(
        body,
        out_shape=jax.ShapeDtypeStruct((M, N), jnp.float32),
        in_specs=[
            pl.BlockSpec(memory_space=pltpu.VMEM),
            pl.BlockSpec(memory_space=pltpu.VMEM),
        ],
        out_specs=pl.BlockSpec(memory_space=pltpu.VMEM),
        scratch_shapes=[
            pltpu.VMEM((768, N), _BF16),
            pltpu.VMEM((R - 1, JR, NH), _BF16),
            pltpu.VMEM((JR, NH), _BF16),
            pltpu.VMEM((R - 1, JR, NH), _BF16),
            pltpu.VMEM((JR, NH), _BF16),
            pltpu.VMEM((C - 1, SR, NH), _BF16),
            pltpu.VMEM((SR, NH), _BF16),
            pltpu.VMEM((C - 1, SR, NH), _BF16),
            pltpu.VMEM((SR, NH), _BF16),
            pltpu.VMEM((C - 1, SR, NH), _BF16),
            pltpu.VMEM((SR, NH), _BF16),
            pltpu.VMEM((C - 1, SR, NH), _BF16),
            pltpu.VMEM((SR, NH), _BF16),
            pltpu.VMEM((R - 1, JR, NH), _BF16),
            pltpu.VMEM((JR, NH), _BF16),
            pltpu.VMEM((R - 1, JR, NH), _BF16),
            pltpu.VMEM((JR, NH), _BF16),
            pltpu.SemaphoreType.DMA((R - 1,)),
            pltpu.SemaphoreType.DMA((R - 1,)),
            pltpu.SemaphoreType.DMA((R - 1,)),
            pltpu.SemaphoreType.DMA((R - 1,)),
            pltpu.SemaphoreType.DMA((C - 1,)),
            pltpu.SemaphoreType.DMA((C - 1,)),
            pltpu.SemaphoreType.DMA((C - 1,)),
            pltpu.SemaphoreType.DMA((C - 1,)),
            pltpu.SemaphoreType.DMA((C - 1,)),
            pltpu.SemaphoreType.DMA((C - 1,)),
            pltpu.SemaphoreType.DMA((C - 1,)),
            pltpu.SemaphoreType.DMA((C - 1,)),
            pltpu.SemaphoreType.DMA((R - 1,)),
            pltpu.SemaphoreType.DMA((R - 1,)),
            pltpu.SemaphoreType.DMA((R - 1,)),
            pltpu.SemaphoreType.DMA((R - 1,)),
        ],
        compiler_params=pltpu.CompilerParams(collective_id=0),
    )(A, B)


# device time: 85180 ns/iter; 3.8655x vs baseline; 1.2157x over previous
import jax
import jax.numpy as jnp
from jax import lax
from jax.experimental import pallas as pl
from jax.experimental.pallas import tpu as pltpu

P = 32
R = 8
C = 4
M = 1536
N = 1536
NH = N // 2
JR = M // R
SR = JR // C
HR = JR // 2

_MESH = pl.DeviceIdType.MESH
_BF16 = jnp.bfloat16
_F32 = jnp.float32


def _q_of(v):
    y = jnp.where(v == 0, 0, jnp.where(v <= 4, v - 1, 8 - v))
    x = jnp.where((v >= 1) & (v <= 4), 1, 0)
    return 2 * y + lax.rem(x + y, 2)


def _k_of(q):
    y = q // 2
    x = lax.rem(q + y, 2)
    return jnp.where(x == 1, y + 1, jnp.where(y == 0, 0, 8 - y))


def kernel(A, B):
    def body(a_ref, b_ref, out_ref, b_bf,
             p1p_land, p1p_stage, p1m_land, p1m_stage,
             p2ap_land, p2ap_stage, p2am_land, p2am_stage,
             p2bp_land, p2bp_stage, p2bm_land, p2bm_stage,
             p3p_land, p3p_stage, p3m_land, p3m_stage,
             p1p_ss, p1p_rs, p1m_ss, p1m_rs,
             p2ap_ss, p2ap_rs, p2am_ss, p2am_rs,
             p2bp_ss, p2bp_rs, p2bm_ss, p2bm_rs,
             p3p_ss, p3p_rs, p3m_ss, p3m_rs):
        my = lax.axis_index("i")
        r = my // R
        q = my % R
        k = _k_of(q)
        row_right = r * R + _q_of(lax.rem(k + 1, R))
        row_left = r * R + _q_of(lax.rem(k + R - 1, R))
        col_next = lax.rem(r + 1, C) * R + q
        col_prev = lax.rem(r + C - 1, C) * R + q

        lo = slice(0, NH)
        hi = slice(NH, N)

        barrier = pltpu.get_barrier_semaphore()
        for nbr in (row_left, row_right, col_prev, col_next):
            pl.semaphore_signal(
                barrier, inc=1, device_id=(nbr,), device_id_type=_MESH,
            )
        pl.semaphore_wait(barrier, 4)

        def copy(src, dst, ssem, rsem, dev):
            return pltpu.make_async_remote_copy(
                src_ref=src, dst_ref=dst, send_sem=ssem, recv_sem=rsem,
                device_id=(dev,), device_id_type=_MESH,
            )

        def p1_desc(s, u, plus):
            rows = pl.ds(u * HR, HR)
            if plus:
                return copy(p1p_stage.at[rows], p1p_land.at[s, rows],
                            p1p_ss.at[s, u], p1p_rs.at[s, u], row_right)
            return copy(p1m_stage.at[rows], p1m_land.at[s, rows],
                        p1m_ss.at[s, u], p1m_rs.at[s, u], row_left)

        def p2a_desc(s, plus):
            if plus:
                return copy(p2ap_stage, p2ap_land.at[s], p2ap_ss.at[s],
                            p2ap_rs.at[s], col_next)
            return copy(p2am_stage, p2am_land.at[s], p2am_ss.at[s],
                        p2am_rs.at[s], col_prev)

        def p2b_desc(t, plus):
            if plus:
                return copy(p2bp_stage if t == 0 else p2bp_land.at[t - 1],
                            p2bp_land.at[t], p2bp_ss.at[t], p2bp_rs.at[t],
                            col_next)
            return copy(p2bm_stage if t == 0 else p2bm_land.at[t - 1],
                        p2bm_land.at[t], p2bm_ss.at[t], p2bm_rs.at[t],
                        col_prev)

        def p3_desc(t, u, plus):
            rows = pl.ds(u * HR, HR)
            if plus:
                src = (p3p_stage.at[rows] if t == 0
                       else p3p_land.at[t - 1, rows])
                return copy(src, p3p_land.at[t, rows],
                            p3p_ss.at[t, u], p3p_rs.at[t, u], row_right)
            src = (p3m_stage.at[rows] if t == 0
                   else p3m_land.at[t - 1, rows])
            return copy(src, p3m_land.at[t, rows],
                        p3m_ss.at[t, u], p3m_rs.at[t, u], row_left)

        b_bf[...] = b_ref[...].astype(_BF16)
        ds_q = pl.ds(q * JR, JR)
        out_ref[ds_q, :] = jnp.dot(
            a_ref[ds_q, :].astype(_BF16), b_bf[...],
            preferred_element_type=_F32,
        )
        p1p_stage[...] = out_ref[ds_q, lo].astype(_BF16)
        p1m_stage[...] = out_ref[ds_q, hi].astype(_BF16)
        for u in (0, 1):
            p1_desc(0, u, True).start()
            p1_desc(0, u, False).start()
        for d in range(1, R):
            j = lax.rem(q + d, R)
            ds_j = pl.ds(j * JR, JR)
            out_ref[ds_j, :] = jnp.dot(
                a_ref[ds_j, :].astype(_BF16), b_bf[...],
                preferred_element_type=_F32,
            )

        for s in range(R - 1):
            for u in (0, 1):
                for plus in (True, False):
                    d = p1_desc(s, u, plus)
                    d.wait_recv()
                    if plus:
                        jj = _q_of(lax.rem(k - s - 1 + R, R))
                        half, land, stage = lo, p1p_land, p1p_stage
                    else:
                        jj = _q_of(lax.rem(k + s + 1, R))
                        half, land, stage = hi, p1m_land, p1m_stage
                    idx = pl.ds(jj * JR + u * HR, HR)
                    new = out_ref[idx, half] + (
                        land[s, u * HR:(u + 1) * HR].astype(_F32))
                    out_ref[idx, half] = new
                    d.wait_send()
                    if s < R - 2:
                        stage[u * HR:(u + 1) * HR] = new.astype(_BF16)
                        p1_desc(s + 1, u, plus).start()

        jp_own = _q_of(lax.rem(k + 1, R))
        jm_own = _q_of(lax.rem(k + R - 1, R))
        base_p = jp_own * JR
        base_m = jm_own * JR

        p2ap_stage[...] = out_ref[pl.ds(base_p + r * SR, SR), lo].astype(_BF16)
        p2am_stage[...] = out_ref[pl.ds(base_m + r * SR, SR), hi].astype(_BF16)
        p2a_desc(0, True).start()
        p2a_desc(0, False).start()
        for s in range(C - 1):
            for plus in (True, False):
                d = p2a_desc(s, plus)
                d.wait_recv()
                if plus:
                    cc = lax.rem(r - s - 1 + C, C)
                    idx = pl.ds(base_p + cc * SR, SR)
                    half, land, stage = lo, p2ap_land, p2ap_stage
                else:
                    cc = lax.rem(r + s + 1, C)
                    idx = pl.ds(base_m + cc * SR, SR)
                    half, land, stage = hi, p2am_land, p2am_stage
                new = out_ref[idx, half] + land[s].astype(_F32)
                out_ref[idx, half] = new
                d.wait_send()
                if s < C - 2:
                    stage[...] = new.astype(_BF16)
                    p2a_desc(s + 1, plus).start()

        cp_own = lax.rem(r + 1, C)
        cm_own = lax.rem(r + C - 1, C)

        p2bp_stage[...] = out_ref[
            pl.ds(base_p + cp_own * SR, SR), lo].astype(_BF16)
        p2bm_stage[...] = out_ref[
            pl.ds(base_m + cm_own * SR, SR), hi].astype(_BF16)
        p2b_desc(0, True).start()
        p2b_desc(0, False).start()
        for t in range(C - 1):
            for plus in (True, False):
                d = p2b_desc(t, plus)
                d.wait_recv()
                if t < C - 2:
                    p2b_desc(t + 1, plus).start()
                if plus:
                    cc = lax.rem(r - t + C, C)
                    out_ref[pl.ds(base_p + cc * SR, SR), lo] = (
                        p2bp_land[t].astype(_F32))
                else:
                    cc = lax.rem(r + t, C)
                    out_ref[pl.ds(base_m + cc * SR, SR), hi] = (
                        p2bm_land[t].astype(_F32))

        p3p_stage[...] = out_ref[pl.ds(base_p, JR), lo].astype(_BF16)
        p3m_stage[...] = out_ref[pl.ds(base_m, JR), hi].astype(_BF16)
        for u in (0, 1):
            p3_desc(0, u, True).start()
            p3_desc(0, u, False).start()
        for t in range(R - 1):
            for u in (0, 1):
                for plus in (True, False):
                    d = p3_desc(t, u, plus)
                    d.wait_recv()
                    if t < R - 2:
                        p3_desc(t + 1, u, plus).start()
                    us = slice(u * HR, (u + 1) * HR)
                    if plus:
                        jj = _q_of(lax.rem(k - t + R, R))
                        out_ref[pl.ds(jj * JR + u * HR, HR), lo] = jnp.maximum(
                            p3p_land[t, us].astype(_F32), 0.0)
                    else:
                        jj = _q_of(lax.rem(k + t, R))
                        out_ref[pl.ds(jj * JR + u * HR, HR), hi] = jnp.maximum(
                            p3m_land[t, us].astype(_F32), 0.0)

        out_ref[pl.ds(base_p, JR), lo] = jnp.maximum(
            out_ref[pl.ds(base_p, JR), lo], 0.0)
        out_ref[pl.ds(base_m, JR), hi] = jnp.maximum(
            out_ref[pl.ds(base_m, JR), hi], 0.0)

        for t in range(C - 1):
            p2b_desc(t, True).wait_send()
            p2b_desc(t, False).wait_send()
        for t in range(R - 1):
            for u in (0, 1):
                p3_desc(t, u, True).wait_send()
                p3_desc(t, u, False).wait_send()

    return pl.pallas_call(
        body,
        out_shape=jax.ShapeDtypeStruct((M, N), jnp.float32),
        in_specs=[
            pl.BlockSpec(memory_space=pltpu.VMEM),
            pl.BlockSpec(memory_space=pltpu.VMEM),
        ],
        out_specs=pl.BlockSpec(memory_space=pltpu.VMEM),
        scratch_shapes=[
            pltpu.VMEM((768, N), _BF16),
            pltpu.VMEM((R - 1, JR, NH), _BF16),
            pltpu.VMEM((JR, NH), _BF16),
            pltpu.VMEM((R - 1, JR, NH), _BF16),
            pltpu.VMEM((JR, NH), _BF16),
            pltpu.VMEM((C - 1, SR, NH), _BF16),
            pltpu.VMEM((SR, NH), _BF16),
            pltpu.VMEM((C - 1, SR, NH), _BF16),
            pltpu.VMEM((SR, NH), _BF16),
            pltpu.VMEM((C - 1, SR, NH), _BF16),
            pltpu.VMEM((SR, NH), _BF16),
            pltpu.VMEM((C - 1, SR, NH), _BF16),
            pltpu.VMEM((SR, NH), _BF16),
            pltpu.VMEM((R - 1, JR, NH), _BF16),
            pltpu.VMEM((JR, NH), _BF16),
            pltpu.VMEM((R - 1, JR, NH), _BF16),
            pltpu.VMEM((JR, NH), _BF16),
            pltpu.SemaphoreType.DMA((R - 1, 2)),
            pltpu.SemaphoreType.DMA((R - 1, 2)),
            pltpu.SemaphoreType.DMA((R - 1, 2)),
            pltpu.SemaphoreType.DMA((R - 1, 2)),
            pltpu.SemaphoreType.DMA((C - 1,)),
            pltpu.SemaphoreType.DMA((C - 1,)),
            pltpu.SemaphoreType.DMA((C - 1,)),
            pltpu.SemaphoreType.DMA((C - 1,)),
            pltpu.SemaphoreType.DMA((C - 1,)),
            pltpu.SemaphoreType.DMA((C - 1,)),
            pltpu.SemaphoreType.DMA((C - 1,)),
            pltpu.SemaphoreType.DMA((C - 1,)),
            pltpu.SemaphoreType.DMA((R - 1, 2)),
            pltpu.SemaphoreType.DMA((R - 1, 2)),
            pltpu.SemaphoreType.DMA((R - 1, 2)),
            pltpu.SemaphoreType.DMA((R - 1, 2)),
        ],
        compiler_params=pltpu.CompilerParams(collective_id=0),
    )(A, B)
